# baseline (speedup 1.0000x reference)
import jax
import jax.numpy as jnp
from jax.experimental import pallas as pl
from jax.experimental.pallas import tpu as pltpu

_VMEM_LIMIT_BYTES = 60000 * 1024


def _mlp_kernel(x_ref, w0_ref, w1_ref, w2_ref, w3_ref,
                b0_ref, b1_ref, b2_ref, b3_ref, o_ref,
                act_a, act_b, wbf):
    l = pl.program_id(0)
    i = pl.program_id(1)
    tm = x_ref.shape[0]
    rows = pl.ds(i * tm, tm)

    def layer(src, w_ref, b_ref, dst_write):
        @pl.when(i == 0)
        def _():
            wbf[...] = w_ref[...].astype(jnp.bfloat16)

        acc = jnp.dot(src, wbf[...], preferred_element_type=jnp.float32)
        dst_write(jnp.maximum(acc + b_ref[...], 0.0))

    @pl.when(l == 0)
    def _():
        layer(x_ref[...].astype(jnp.bfloat16), w0_ref, b0_ref,
              lambda a: act_a.__setitem__((rows, slice(None)),
                                          a.astype(jnp.bfloat16)))

    @pl.when(l == 1)
    def _():
        layer(act_a[rows, :], w1_ref, b1_ref,
              lambda a: act_b.__setitem__((rows, slice(None)),
                                          a.astype(jnp.bfloat16)))

    @pl.when(l == 2)
    def _():
        layer(act_b[rows, :], w2_ref, b2_ref,
              lambda a: act_a.__setitem__((rows, slice(None)),
                                          a.astype(jnp.bfloat16)))

    @pl.when(l == 3)
    def _():
        layer(act_a[rows, :], w3_ref, b3_ref,
              lambda a: o_ref.__setitem__((Ellipsis,), a))


def _fused_mlp(h, ws, bs, tm):
    M, F = h.shape
    nt = M // tm
    L = 4
    w_spec = pl.BlockSpec((F, F), lambda l, i: (0, 0))
    b_spec = pl.BlockSpec((1, F), lambda l, i: (0, 0))
    x_spec = pl.BlockSpec((tm, F),
                          lambda l, i: (jnp.where(l == 0, i, nt - 1), 0))
    o_spec = pl.BlockSpec((tm, F),
                          lambda l, i: (jnp.where(l == L - 1, i, 0), 0))
    return pl.pallas_call(
        _mlp_kernel,
        out_shape=jax.ShapeDtypeStruct((M, F), jnp.float32),
        grid_spec=pltpu.PrefetchScalarGridSpec(
            num_scalar_prefetch=0,
            grid=(L, nt),
            in_specs=[x_spec] + [w_spec] * 4 + [b_spec] * 4,
            out_specs=o_spec,
            scratch_shapes=[
                pltpu.VMEM((M, F), jnp.bfloat16),
                pltpu.VMEM((M, F), jnp.bfloat16),
                pltpu.VMEM((F, F), jnp.bfloat16),
            ],
        ),
        compiler_params=pltpu.CompilerParams(
            dimension_semantics=("arbitrary", "arbitrary"),
            vmem_limit_bytes=_VMEM_LIMIT_BYTES,
        ),
        cost_estimate=pl.CostEstimate(
            flops=2 * M * F * F * L,
            transcendentals=0,
            bytes_accessed=4 * (M * F + L * F * F + L * F + M * F),
        ),
    )(h, *ws, *bs)


def kernel(x, w0, b0, w1, b1, w2, b2, w3, b3):
    bcz, seq_len, in_f = x.shape
    h = x.reshape(-1, in_f)
    M = h.shape[0]
    tm = 1024 if M % 1024 == 0 else (512 if M % 512 == 0 else 256)
    out = _fused_mlp(h, [w0, w1, w2, w3],
                     [b.reshape(1, -1) for b in (b0, b1, b2, b3)], tm)
    return out.reshape(bcz, seq_len, -1)


# speedup vs baseline: 1.0593x; 1.0593x over previous
import jax
import jax.numpy as jnp
from jax.experimental import pallas as pl
from jax.experimental.pallas import tpu as pltpu

_VMEM_LIMIT_BYTES = 52 * 1024 * 1024


def _mlp_kernel(x_ref, w0_ref, w1_ref, w2_ref, w3_ref,
                b0_ref, b1_ref, b2_ref, b3_ref, o_ref,
                s1, s2, s3, sems):
    i = pl.program_id(0)

    @pl.when(i == 0)
    def _():
        pltpu.make_async_copy(w1_ref, s1, sems.at[0]).start()
        pltpu.make_async_copy(w2_ref, s2, sems.at[1]).start()
        pltpu.make_async_copy(w3_ref, s3, sems.at[2]).start()

    h = jnp.maximum(
        jnp.dot(x_ref[...], w0_ref[...], preferred_element_type=jnp.float32)
        + b0_ref[...], 0.0)

    @pl.when(i == 0)
    def _():
        pltpu.make_async_copy(w1_ref, s1, sems.at[0]).wait()

    h = jnp.maximum(
        jnp.dot(h, s1[...], preferred_element_type=jnp.float32)
        + b1_ref[...], 0.0)

    @pl.when(i == 0)
    def _():
        pltpu.make_async_copy(w2_ref, s2, sems.at[1]).wait()

    h = jnp.maximum(
        jnp.dot(h, s2[...], preferred_element_type=jnp.float32)
        + b2_ref[...], 0.0)

    @pl.when(i == 0)
    def _():
        pltpu.make_async_copy(w3_ref, s3, sems.at[2]).wait()

    o_ref[...] = jnp.maximum(
        jnp.dot(h, s3[...], preferred_element_type=jnp.float32)
        + b3_ref[...], 0.0)


def _fused_mlp(h, ws, bs, tm):
    M, F = h.shape
    row_spec = pl.BlockSpec((tm, F), lambda i: (i, 0))
    w0_spec = pl.BlockSpec((F, F), lambda i: (0, 0))
    any_spec = pl.BlockSpec(memory_space=pl.ANY)
    b_spec = pl.BlockSpec((1, F), lambda i: (0, 0))
    return pl.pallas_call(
        _mlp_kernel,
        out_shape=jax.ShapeDtypeStruct((M, F), jnp.float32),
        grid_spec=pltpu.PrefetchScalarGridSpec(
            num_scalar_prefetch=0,
            grid=(M // tm,),
            in_specs=[row_spec, w0_spec, any_spec, any_spec, any_spec]
                     + [b_spec] * 4,
            out_specs=row_spec,
            scratch_shapes=[
                pltpu.VMEM((F, F), jnp.float32),
                pltpu.VMEM((F, F), jnp.float32),
                pltpu.VMEM((F, F), jnp.float32),
                pltpu.SemaphoreType.DMA((3,)),
            ],
        ),
        compiler_params=pltpu.CompilerParams(
            dimension_semantics=("arbitrary",),
            vmem_limit_bytes=_VMEM_LIMIT_BYTES,
        ),
        cost_estimate=pl.CostEstimate(
            flops=2 * M * F * F * 4,
            transcendentals=0,
            bytes_accessed=4 * (M * F + 4 * F * F + 4 * F + M * F),
        ),
    )(h, *ws, *bs)


def kernel(x, w0, b0, w1, b1, w2, b2, w3, b3):
    bcz, seq_len, in_f = x.shape
    h = x.reshape(-1, in_f)
    M = h.shape[0]
    tm = 1024 if M % 1024 == 0 else (512 if M % 512 == 0 else 256)
    out = _fused_mlp(h, [w0, w1, w2, w3],
                     [b.reshape(1, -1) for b in (b0, b1, b2, b3)], tm)
    return out.reshape(bcz, seq_len, -1)


# speedup vs baseline: 1.0964x; 1.0350x over previous
import jax
import jax.numpy as jnp
from jax.experimental import pallas as pl
from jax.experimental.pallas import tpu as pltpu

_VMEM_LIMIT_BYTES = 60000 * 1024


def _mlp_kernel(x_ref, w0_ref, w1_ref, w2_ref, w3_ref,
                b0_ref, b1_ref, b2_ref, b3_ref, o_ref):
    tm = x_ref.shape[0]
    nc = 4
    cm = tm // nc
    hs = [x_ref[pl.ds(c * cm, cm), :] for c in range(nc)]
    for w_ref, b_ref, last in (
        (w0_ref, b0_ref, False),
        (w1_ref, b1_ref, False),
        (w2_ref, b2_ref, False),
        (w3_ref, b3_ref, True),
    ):
        b = b_ref[...]
        nxt = []
        for c in range(nc):
            acc = jnp.dot(hs[c], w_ref[...], preferred_element_type=jnp.float32)
            a = jnp.maximum(acc + b, 0.0)
            if last:
                o_ref[pl.ds(c * cm, cm), :] = a
            else:
                nxt.append(a)
        hs = nxt


def _fused_mlp(h, ws, bs, tm):
    M, F = h.shape
    row_spec = pl.BlockSpec((tm, F), lambda i: (i, 0))
    w_spec = pl.BlockSpec((F, F), lambda i: (0, 0))
    b_spec = pl.BlockSpec((1, F), lambda i: (0, 0))
    return pl.pallas_call(
        _mlp_kernel,
        out_shape=jax.ShapeDtypeStruct((M, F), jnp.float32),
        grid=(M // tm,),
        in_specs=[row_spec] + [w_spec] * 4 + [b_spec] * 4,
        out_specs=row_spec,
        compiler_params=pltpu.CompilerParams(
            dimension_semantics=("arbitrary",),
            vmem_limit_bytes=_VMEM_LIMIT_BYTES,
        ),
        cost_estimate=pl.CostEstimate(
            flops=2 * M * F * F * 4,
            transcendentals=0,
            bytes_accessed=4 * (M * F + 4 * F * F + 4 * F + M * F),
        ),
    )(h, *ws, *bs)


def kernel(x, w0, b0, w1, b1, w2, b2, w3, b3):
    bcz, seq_len, in_f = x.shape
    h = x.reshape(-1, in_f)
    M = h.shape[0]
    tm = 2048 if M % 2048 == 0 else (512 if M % 512 == 0 else 256)
    out = _fused_mlp(h, [w0, w1, w2, w3],
                     [b.reshape(1, -1) for b in (b0, b1, b2, b3)], tm)
    return out.reshape(bcz, seq_len, -1)
